# Initial kernel scaffold; baseline (speedup 1.0000x reference)
import jax
import jax.numpy as jnp
from jax.experimental import pallas as pl


def kernel(x, unused, e, W, b, rms_weight, rms_bias):
    raise NotImplementedError("write your pallas kernel here")


# speedup vs baseline: 1.5426x; 1.5426x over previous
import functools

import jax
import jax.numpy as jnp
from jax import lax
from jax.experimental import pallas as pl
from jax.experimental.pallas import tpu as pltpu
from jax.experimental.pallas import tpu_sc as plsc

H = 128
N = 10000
E = 320000
EPS = 1e-5

NT = 32
RPT = 313
NPAD = NT * RPT
C = 1280
NCHUNK = E // C
VPC = C // 16
G = 16


def _relu_linear(x, Wt, b2):
    def body(x_ref, wt_ref, b_ref, y_ref):
        y_ref[...] = jnp.maximum(
            jnp.dot(x_ref[...], wt_ref[...],
                    preferred_element_type=jnp.float32) + b_ref[...], 0.0)

    grid = 10
    blk = N // grid
    return pl.pallas_call(
        body,
        grid=(grid,),
        in_specs=[
            pl.BlockSpec((blk, H), lambda i: (i, 0)),
            pl.BlockSpec((H, H), lambda i: (0, 0)),
            pl.BlockSpec((1, H), lambda i: (0, 0)),
        ],
        out_specs=pl.BlockSpec((blk, H), lambda i: (i, 0)),
        out_shape=jax.ShapeDtypeStruct((N, H), jnp.float32),
    )(x, Wt, b2)


def _make_segmax():
    mesh = plsc.VectorSubcoreMesh(core_axis_name="c", subcore_axis_name="s")

    @functools.partial(
        pl.kernel,
        mesh=mesh,
        out_type=jax.ShapeDtypeStruct((NPAD * H,), jnp.float32),
        scratch_types=[
            pltpu.VMEM(((RPT + 1) * H,), jnp.float32),
            pltpu.VMEM((C,), jnp.int32),
            pltpu.VMEM((C,), jnp.int32),
            pltpu.VMEM((C + G,), jnp.int32),
            pltpu.VMEM((C + G,), jnp.int32),
            pltpu.VMEM((G, H), jnp.float32),
            pltpu.SemaphoreType.DMA,
        ],
        compiler_params=pltpu.CompilerParams(needs_layout_passes=False),
    )
    def segmax(y_hbm, src_hbm, dst_hbm, out_hbm,
               acc, dstv, srcv, cdst, csrc, rows, sem):
        wid = lax.axis_index("s") * 2 + lax.axis_index("c")
        lo = wid * RPT
        col = lax.iota(jnp.int32, 16)
        zero16f = jnp.zeros((16,), jnp.float32)
        zero16i = jnp.zeros((16,), jnp.int32)
        dump16 = jnp.full((16,), RPT, jnp.int32)

        def init_body(i, carry):
            acc[pl.ds(i * 16, 16)] = zero16f
            return carry
        lax.fori_loop(0, (RPT + 1) * H // 16, init_body, 0)
        for j in range((C + G) // 16):
            csrc[pl.ds(j * 16, 16)] = zero16i
            cdst[pl.ds(j * 16, 16)] = dump16

        def chunk_body(ci, carry):
            pltpu.sync_copy(dst_hbm.at[pl.ds(ci * C, C)], dstv)
            pltpu.sync_copy(src_hbm.at[pl.ds(ci * C, C)], srcv)

            def scan_body(j, cnt):
                d16 = dstv[pl.ds(j * 16, 16)]
                s16 = srcv[pl.ds(j * 16, 16)]
                dl = d16 - lo
                msk = (dl >= 0) & (dl < RPT)
                pref = plsc.cumsum(msk.astype(jnp.int32))
                pos = pref + (cnt - 1)
                plsc.store_scatter(cdst, [pos], dl, mask=msk)
                plsc.store_scatter(csrc, [pos], s16, mask=msk)
                return cnt + jnp.max(pref)
            cnt = lax.fori_loop(0, VPC, scan_body, jnp.int32(0))

            cdst[pl.ds(cnt, 16)] = dump16

            ng = (cnt + (G - 1)) // G

            def group_body(g, carry2):
                pltpu.async_copy(
                    y_hbm.at[csrc.at[pl.ds(g * G, G)]], rows, sem).wait()
                for r in range(G):
                    d_spl = plsc.load_gather(
                        cdst, [jnp.full((16,), g * G + r, jnp.int32)])
                    base = d_spl * H
                    for q in range(8):
                        idx = base + (col + 16 * q)
                        cur = plsc.load_gather(acc, [idx])
                        plsc.store_scatter(
                            acc, [idx],
                            jnp.maximum(cur, rows[r, pl.ds(16 * q, 16)]))
                return carry2
            lax.fori_loop(0, ng, group_body, 0)
            return carry
        lax.fori_loop(0, NCHUNK, chunk_body, 0)

        pltpu.sync_copy(acc.at[pl.ds(0, RPT * H)],
                        out_hbm.at[pl.ds(lo * H, RPT * H)])

    return segmax


_segmax = _make_segmax()


def _finalize(x, mm, w2, b2):
    def body(x_ref, mm_ref, w_ref, b_ref, o_ref):
        h = x_ref[...] + mm_ref[...]
        ms = jnp.mean(h * h, axis=-1, keepdims=True)
        o_ref[...] = h * lax.rsqrt(ms + EPS) * w_ref[...] + b_ref[...]

    grid = 10
    blk = N // grid
    return pl.pallas_call(
        body,
        grid=(grid,),
        in_specs=[
            pl.BlockSpec((blk, H), lambda i: (i, 0)),
            pl.BlockSpec((blk, H), lambda i: (i, 0)),
            pl.BlockSpec((1, H), lambda i: (0, 0)),
            pl.BlockSpec((1, H), lambda i: (0, 0)),
        ],
        out_specs=pl.BlockSpec((blk, H), lambda i: (i, 0)),
        out_shape=jax.ShapeDtypeStruct((N, H), jnp.float32),
    )(x, mm, w2, b2)


def kernel(x, unused, e, W, b, rms_weight, rms_bias):
    y = _relu_linear(x, W.T, b.reshape(1, H))
    src = e[:, 0]
    dst = e[:, 1]
    mm_flat = _segmax(y, src, dst)
    mm = mm_flat.reshape(NPAD, H)[:N]
    return _finalize(x, mm, rms_weight.reshape(1, H), rms_bias.reshape(1, H))
